# initial kernel scaffold (unmeasured)
import jax
import jax.numpy as jnp
from jax import lax
from jax.experimental import pallas as pl
from jax.experimental.pallas import tpu as pltpu


def kernel(
    x,
):
    def body(*refs):
        pass

    out_shape = jax.ShapeDtypeStruct(..., jnp.float32)
    return pl.pallas_call(body, out_shape=out_shape)(...)


# baseline (device time: 15437 ns/iter reference)
import jax
import jax.numpy as jnp
from jax import lax
from jax.experimental import pallas as pl
from jax.experimental.pallas import tpu as pltpu

N_DEV = 8
N_ROUNDS = 3


def kernel(x):
    m_per, n = x.shape

    def body(x_ref, out_ref, acc_ref, comm_ref, send_sems, recv_sems):
        my = lax.axis_index("i")

        barrier_sem = pltpu.get_barrier_semaphore()
        for r in range(N_ROUNDS):
            partner = my ^ (1 << r)
            pl.semaphore_signal(
                barrier_sem, inc=1,
                device_id=(partner,), device_id_type=pl.DeviceIdType.MESH,
            )
        pl.semaphore_wait(barrier_sem, N_ROUNDS)

        xv = x_ref[:, :]
        row_ids = lax.broadcasted_iota(jnp.int32, (m_per, n), 0)
        local_max = jnp.max(xv, axis=0)
        is_max = xv == local_max[None, :]
        local_idx = jnp.min(
            jnp.where(is_max, row_ids, jnp.int32(m_per)), axis=0
        ).astype(jnp.float32) + my.astype(jnp.float32) * jnp.float32(m_per)
        acc_ref[0, :] = local_max
        acc_ref[1, :] = local_idx

        for r in range(N_ROUNDS):
            partner = my ^ (1 << r)
            rdma = pltpu.make_async_remote_copy(
                src_ref=acc_ref,
                dst_ref=comm_ref.at[r],
                send_sem=send_sems.at[r],
                recv_sem=recv_sems.at[r],
                device_id=(partner,),
                device_id_type=pl.DeviceIdType.MESH,
            )
            rdma.start()
            rdma.wait()

            v, i = acc_ref[0, :], acc_ref[1, :]
            ov, oi = comm_ref[r, 0, :], comm_ref[r, 1, :]
            take = (ov > v) | ((ov == v) & (oi < i))
            acc_ref[0, :] = jnp.where(take, ov, v)
            acc_ref[1, :] = jnp.where(take, oi, i)

        out_ref[:, :] = acc_ref[:, :]

    return pl.pallas_call(
        body,
        out_shape=jax.ShapeDtypeStruct((2, n), jnp.float32),
        in_specs=[pl.BlockSpec(memory_space=pltpu.VMEM)],
        out_specs=pl.BlockSpec(memory_space=pltpu.VMEM),
        scratch_shapes=[
            pltpu.VMEM((2, n), jnp.float32),
            pltpu.VMEM((N_ROUNDS, 2, n), jnp.float32),
            pltpu.SemaphoreType.DMA((N_ROUNDS,)),
            pltpu.SemaphoreType.DMA((N_ROUNDS,)),
        ],
        compiler_params=pltpu.CompilerParams(collective_id=0),
    )(x)


# device time: 12416 ns/iter; 1.2433x vs baseline; 1.2433x over previous
import jax
import jax.numpy as jnp
from jax import lax
from jax.experimental import pallas as pl
from jax.experimental.pallas import tpu as pltpu

N_DEV = 8


def kernel(x):
    m_per, n = x.shape

    def body(x_ref, out_ref, comm_ref, send_sems, recv_sems):
        my = lax.axis_index("i")

        barrier_sem = pltpu.get_barrier_semaphore()
        for d in range(1, N_DEV):
            peer = lax.rem(my + d, N_DEV)
            pl.semaphore_signal(
                barrier_sem, inc=1,
                device_id=(peer,), device_id_type=pl.DeviceIdType.MESH,
            )
        pl.semaphore_wait(barrier_sem, N_DEV - 1)

        xv = x_ref[:, :]
        row_ids = lax.broadcasted_iota(jnp.int32, (m_per, n), 0)
        local_max = jnp.max(xv, axis=0)
        is_max = xv == local_max[None, :]
        local_idx = jnp.min(
            jnp.where(is_max, row_ids, jnp.int32(m_per)), axis=0
        ).astype(jnp.float32) + my.astype(jnp.float32) * jnp.float32(m_per)
        comm_ref[my, 0, :] = local_max
        comm_ref[my, 1, :] = local_idx

        for d in range(1, N_DEV):
            peer = lax.rem(my + d, N_DEV)
            rdma = pltpu.make_async_remote_copy(
                src_ref=comm_ref.at[my],
                dst_ref=comm_ref.at[my],
                send_sem=send_sems.at[peer],
                recv_sem=recv_sems.at[my],
                device_id=(peer,),
                device_id_type=pl.DeviceIdType.MESH,
            )
            rdma.start()

        for d in range(1, N_DEV):
            peer = lax.rem(my + d, N_DEV)
            recv = pltpu.make_async_remote_copy(
                src_ref=comm_ref.at[peer],
                dst_ref=comm_ref.at[peer],
                send_sem=send_sems.at[peer],
                recv_sem=recv_sems.at[peer],
                device_id=(peer,),
                device_id_type=pl.DeviceIdType.MESH,
            )
            recv.wait_recv()

        vals = comm_ref[:, 0, :]
        idxs = comm_ref[:, 1, :]
        gmax = jnp.max(vals, axis=0)
        gidx = jnp.min(
            jnp.where(vals == gmax[None, :], idxs, jnp.float32(jnp.inf)), axis=0
        )
        out_ref[0, :] = gmax
        out_ref[1, :] = gidx

        for d in range(1, N_DEV):
            peer = lax.rem(my + d, N_DEV)
            snd = pltpu.make_async_remote_copy(
                src_ref=comm_ref.at[my],
                dst_ref=comm_ref.at[my],
                send_sem=send_sems.at[peer],
                recv_sem=recv_sems.at[peer],
                device_id=(peer,),
                device_id_type=pl.DeviceIdType.MESH,
            )
            snd.wait_send()

    return pl.pallas_call(
        body,
        out_shape=jax.ShapeDtypeStruct((2, n), jnp.float32),
        in_specs=[pl.BlockSpec(memory_space=pltpu.VMEM)],
        out_specs=pl.BlockSpec(memory_space=pltpu.VMEM),
        scratch_shapes=[
            pltpu.VMEM((N_DEV, 2, n), jnp.float32),
            pltpu.SemaphoreType.DMA((N_DEV,)),
            pltpu.SemaphoreType.DMA((N_DEV,)),
        ],
        compiler_params=pltpu.CompilerParams(collective_id=0),
    )(x)


# device time: 11894 ns/iter; 1.2979x vs baseline; 1.0439x over previous
import jax
import jax.numpy as jnp
from jax import lax
from jax.experimental import pallas as pl
from jax.experimental.pallas import tpu as pltpu

N_DEV = 8


def kernel(x):
    m_per, n = x.shape

    def body(x_ref, out_ref, comm_ref, send_sems, recv_sems):
        my = lax.axis_index("i")

        barrier_sem = pltpu.get_barrier_semaphore()
        for d in range(1, N_DEV):
            peer = lax.rem(my + d, N_DEV)
            pl.semaphore_signal(
                barrier_sem, inc=1,
                device_id=(peer,), device_id_type=pl.DeviceIdType.MESH,
            )
        pl.semaphore_wait(barrier_sem, N_DEV - 1)

        n_blk = m_per // 8

        def step(b, carry):
            m, bidx = carry
            blk = x_ref[pl.ds(b * 8, 8), :]
            take = blk > m
            return (
                jnp.where(take, blk, m),
                jnp.where(take, b, bidx),
            )

        m0 = jnp.full((8, n), -jnp.inf, jnp.float32)
        b0 = jnp.zeros((8, n), jnp.int32)
        m, bidx = lax.fori_loop(0, n_blk, step, (m0, b0), unroll=8)

        local_max = jnp.max(m, axis=0)
        sub = lax.broadcasted_iota(jnp.int32, (8, n), 0)
        rows = bidx * 8 + sub
        cand = jnp.where(m == local_max[None, :], rows, jnp.int32(m_per))
        local_idx = jnp.min(cand, axis=0).astype(jnp.float32) + (
            my.astype(jnp.float32) * jnp.float32(m_per)
        )
        comm_ref[my, 0, :] = local_max
        comm_ref[my, 1, :] = local_idx

        for d in range(1, N_DEV):
            peer = lax.rem(my + d, N_DEV)
            rdma = pltpu.make_async_remote_copy(
                src_ref=comm_ref.at[my],
                dst_ref=comm_ref.at[my],
                send_sem=send_sems.at[peer],
                recv_sem=recv_sems.at[my],
                device_id=(peer,),
                device_id_type=pl.DeviceIdType.MESH,
            )
            rdma.start()

        for d in range(1, N_DEV):
            peer = lax.rem(my + d, N_DEV)
            recv = pltpu.make_async_remote_copy(
                src_ref=comm_ref.at[peer],
                dst_ref=comm_ref.at[peer],
                send_sem=send_sems.at[peer],
                recv_sem=recv_sems.at[peer],
                device_id=(peer,),
                device_id_type=pl.DeviceIdType.MESH,
            )
            recv.wait_recv()

        vals = comm_ref[:, 0, :]
        idxs = comm_ref[:, 1, :]
        gmax = jnp.max(vals, axis=0)
        gidx = jnp.min(
            jnp.where(vals == gmax[None, :], idxs, jnp.float32(jnp.inf)), axis=0
        )
        out_ref[0, :] = gmax
        out_ref[1, :] = gidx

        for d in range(1, N_DEV):
            peer = lax.rem(my + d, N_DEV)
            snd = pltpu.make_async_remote_copy(
                src_ref=comm_ref.at[my],
                dst_ref=comm_ref.at[my],
                send_sem=send_sems.at[peer],
                recv_sem=recv_sems.at[peer],
                device_id=(peer,),
                device_id_type=pl.DeviceIdType.MESH,
            )
            snd.wait_send()

    return pl.pallas_call(
        body,
        out_shape=jax.ShapeDtypeStruct((2, n), jnp.float32),
        in_specs=[pl.BlockSpec(memory_space=pltpu.VMEM)],
        out_specs=pl.BlockSpec(memory_space=pltpu.VMEM),
        scratch_shapes=[
            pltpu.VMEM((N_DEV, 2, n), jnp.float32),
            pltpu.SemaphoreType.DMA((N_DEV,)),
            pltpu.SemaphoreType.DMA((N_DEV,)),
        ],
        compiler_params=pltpu.CompilerParams(collective_id=0),
    )(x)


# device time: 5209 ns/iter; 2.9635x vs baseline; 2.2834x over previous
import os

import jax
import jax.numpy as jnp
from jax import lax
from jax.experimental import pallas as pl
from jax.experimental.pallas import tpu as pltpu

N_DEV = 8

_VARIANT = os.environ.get("KERNEL_VARIANT", "full")


def kernel(x):
    m_per, n = x.shape

    if _VARIANT == "copyonly":
        def body_copy(x_ref, out_ref):
            out_ref[:, :] = x_ref[:2, :]
        return pl.pallas_call(
            body_copy,
            out_shape=jax.ShapeDtypeStruct((2, n), jnp.float32),
            in_specs=[pl.BlockSpec(memory_space=pltpu.VMEM)],
            out_specs=pl.BlockSpec(memory_space=pltpu.VMEM),
        )(x)

    if _VARIANT == "nocomm":
        def body_nc(x_ref, out_ref):
            my = lax.axis_index("i")
            n_blk = m_per // 8

            def step(b, carry):
                m, bidx = carry
                blk = x_ref[pl.ds(b * 8, 8), :]
                take = blk > m
                return (jnp.where(take, blk, m), jnp.where(take, b, bidx))

            m0 = jnp.full((8, n), -jnp.inf, jnp.float32)
            b0 = jnp.zeros((8, n), jnp.int32)
            m, bidx = lax.fori_loop(0, n_blk, step, (m0, b0), unroll=8)
            local_max = jnp.max(m, axis=0)
            sub = lax.broadcasted_iota(jnp.int32, (8, n), 0)
            rows = bidx * 8 + sub
            cand = jnp.where(m == local_max[None, :], rows, jnp.int32(m_per))
            out_ref[0, :] = local_max
            out_ref[1, :] = jnp.min(cand, axis=0).astype(jnp.float32) + (
                my.astype(jnp.float32) * jnp.float32(m_per)
            )
        return pl.pallas_call(
            body_nc,
            out_shape=jax.ShapeDtypeStruct((2, n), jnp.float32),
            in_specs=[pl.BlockSpec(memory_space=pltpu.VMEM)],
            out_specs=pl.BlockSpec(memory_space=pltpu.VMEM),
        )(x)

    def body(x_ref, out_ref, comm_ref, send_sems, recv_sems):
        my = lax.axis_index("i")

        barrier_sem = pltpu.get_barrier_semaphore()
        for d in range(1, N_DEV):
            peer = lax.rem(my + d, N_DEV)
            pl.semaphore_signal(
                barrier_sem, inc=1,
                device_id=(peer,), device_id_type=pl.DeviceIdType.MESH,
            )
        pl.semaphore_wait(barrier_sem, N_DEV - 1)

        n_blk = m_per // 8

        def step(b, carry):
            m, bidx = carry
            blk = x_ref[pl.ds(b * 8, 8), :]
            take = blk > m
            return (
                jnp.where(take, blk, m),
                jnp.where(take, b, bidx),
            )

        m0 = jnp.full((8, n), -jnp.inf, jnp.float32)
        b0 = jnp.zeros((8, n), jnp.int32)
        m, bidx = lax.fori_loop(0, n_blk, step, (m0, b0), unroll=8)

        local_max = jnp.max(m, axis=0)
        sub = lax.broadcasted_iota(jnp.int32, (8, n), 0)
        rows = bidx * 8 + sub
        cand = jnp.where(m == local_max[None, :], rows, jnp.int32(m_per))
        local_idx = jnp.min(cand, axis=0).astype(jnp.float32) + (
            my.astype(jnp.float32) * jnp.float32(m_per)
        )
        comm_ref[my, 0, :] = local_max
        comm_ref[my, 1, :] = local_idx

        for d in range(1, N_DEV):
            peer = lax.rem(my + d, N_DEV)
            rdma = pltpu.make_async_remote_copy(
                src_ref=comm_ref.at[my],
                dst_ref=comm_ref.at[my],
                send_sem=send_sems.at[peer],
                recv_sem=recv_sems.at[my],
                device_id=(peer,),
                device_id_type=pl.DeviceIdType.MESH,
            )
            rdma.start()

        for d in range(1, N_DEV):
            peer = lax.rem(my + d, N_DEV)
            recv = pltpu.make_async_remote_copy(
                src_ref=comm_ref.at[peer],
                dst_ref=comm_ref.at[peer],
                send_sem=send_sems.at[peer],
                recv_sem=recv_sems.at[peer],
                device_id=(peer,),
                device_id_type=pl.DeviceIdType.MESH,
            )
            recv.wait_recv()

        vals = comm_ref[:, 0, :]
        idxs = comm_ref[:, 1, :]
        gmax = jnp.max(vals, axis=0)
        gidx = jnp.min(
            jnp.where(vals == gmax[None, :], idxs, jnp.float32(jnp.inf)), axis=0
        )
        out_ref[0, :] = gmax
        out_ref[1, :] = gidx

        for d in range(1, N_DEV):
            peer = lax.rem(my + d, N_DEV)
            snd = pltpu.make_async_remote_copy(
                src_ref=comm_ref.at[my],
                dst_ref=comm_ref.at[my],
                send_sem=send_sems.at[peer],
                recv_sem=recv_sems.at[peer],
                device_id=(peer,),
                device_id_type=pl.DeviceIdType.MESH,
            )
            snd.wait_send()

    return pl.pallas_call(
        body,
        out_shape=jax.ShapeDtypeStruct((2, n), jnp.float32),
        in_specs=[pl.BlockSpec(memory_space=pltpu.VMEM)],
        out_specs=pl.BlockSpec(memory_space=pltpu.VMEM),
        scratch_shapes=[
            pltpu.VMEM((N_DEV, 2, n), jnp.float32),
            pltpu.SemaphoreType.DMA((N_DEV,)),
            pltpu.SemaphoreType.DMA((N_DEV,)),
        ],
        compiler_params=pltpu.CompilerParams(collective_id=0),
    )(x)
